# baseline (speedup 1.0000x reference)
import functools

import jax
import jax.numpy as jnp
from jax import lax
from jax.experimental import pallas as pl
from jax.experimental.pallas import tpu as pltpu
from jax.experimental.pallas import tpu_sc as plsc

B = 16384
D = 16
NROW = 1000000
NBLK = NROW // 8
NC = 2
NS = 16
NW = NC * NS
BPW = B // NW
CW = 64
NCH = BPW // CW
OBW = BPW // 8


def _sc_gather(user_indices, item_indices, t_mfu, t_mfi, t_mlu, t_mli):
    mesh = plsc.VectorSubcoreMesh(core_axis_name="c", subcore_axis_name="s")

    @functools.partial(
        pl.kernel,
        out_type=[jax.ShapeDtypeStruct((B // 8, 128), jnp.float32)
                  for _ in range(4)],
        mesh=mesh,
        scratch_types=(
            [pltpu.VMEM((BPW,), jnp.int32) for _ in range(4)]
            + [pltpu.VMEM((CW, 128), jnp.float32) for _ in range(8)]
            + [pltpu.VMEM((OBW, 128), jnp.float32) for _ in range(4)]
            + [pltpu.SemaphoreType.DMA, pltpu.SemaphoreType.DMA]
        ),
        compiler_params=pltpu.CompilerParams(use_tc_tiling_on_sc=True),
    )
    def gather_k(uidx, iidx, tmfu, tmfi, tmlu, tmli,
                 o_mfu, o_mfi, o_mlu, o_mli,
                 uix, iix, ublk, iblk,
                 g0, g1, g2, g3, g4, g5, g6, g7,
                 ob_mfu, ob_mfi, ob_mlu, ob_mli,
                 sem0, sem1):
        wid = lax.axis_index("s") * NC + lax.axis_index("c")
        base = wid * BPW
        pltpu.sync_copy(uidx.at[pl.ds(base, BPW)], uix)
        pltpu.sync_copy(iidx.at[pl.ds(base, BPW)], iix)
        for c in range(BPW // 16):
            sl = pl.ds(c * 16, 16)
            ublk[sl] = jnp.right_shift(uix[sl], 3)
            iblk[sl] = jnp.right_shift(iix[sl], 3)

        gsets = ((g0, g1, g2, g3), (g4, g5, g6, g7))
        sems = (sem0, sem1)
        obufs = (ob_mfu, ob_mfi, ob_mlu, ob_mli)

        def fire(c):
            bufs = gsets[c % 2]
            sem = sems[c % 2]
            hs = []
            for tbl, blk, buf in ((tmfu, ublk, bufs[0]), (tmfi, iblk, bufs[1]),
                                  (tmlu, ublk, bufs[2]), (tmli, iblk, bufs[3])):
                hs.append(pltpu.async_copy(
                    tbl.at[blk.at[pl.ds(c * CW, CW)]], buf, sem))
            return hs

        def extract(c):
            gu, gi, gmu, gmi = gsets[c % 2]

            def body(g, carry):
                uvec = uix[pl.ds(c * CW + g * 16, 16)]
                ivec = iix[pl.ds(c * CW + g * 16, 16)]
                uo = jnp.bitwise_and(uvec, 7) * 16
                io = jnp.bitwise_and(ivec, 7) * 16
                for j in range(16):
                    i = g * 16 + j
                    orow = c * (CW // 8) + g * 2 + (j // 8)
                    osl = pl.ds((j % 8) * 16, D)
                    uoj = uo[j]
                    ioj = io[j]
                    ob_mfu[orow, osl] = gu[i, pl.ds(uoj, D)]
                    ob_mfi[orow, osl] = gi[i, pl.ds(ioj, D)]
                    ob_mlu[orow, osl] = gmu[i, pl.ds(uoj, D)]
                    ob_mli[orow, osl] = gmi[i, pl.ds(ioj, D)]
                return carry

            lax.fori_loop(0, CW // 16, body, 0)

        prev = None
        for c in range(NCH):
            cur = fire(c)
            if prev is not None:
                for h in prev:
                    h.wait()
                extract(c - 1)
            prev = cur
        for h in prev:
            h.wait()
        extract(NCH - 1)

        out_sl = pl.ds(wid * OBW, OBW)
        for ob, o in zip(obufs, (o_mfu, o_mfi, o_mlu, o_mli)):
            pltpu.sync_copy(ob, o.at[out_sl])

    return gather_k(user_indices, item_indices, t_mfu, t_mfi, t_mlu, t_mli)


BB = 2048


def _dense_body(mfu_ref, mfi_ref, mlu_ref, mli_ref, w0t_ref, b0_ref, wp_ref,
                bp_ref, out_ref):
    mf = mfu_ref[...] * mfi_ref[...]
    mlp_vec = jnp.concatenate([mlu_ref[...], mli_ref[...]], axis=1)
    h = jnp.dot(mlp_vec, w0t_ref[...], preferred_element_type=jnp.float32)
    h = jnp.maximum(h + b0_ref[...], 0.0)
    wp = wp_ref[...]
    logit = (jnp.sum(mf * wp[:, :D], axis=1)
             + jnp.sum(h * wp[:, D:], axis=1)
             + bp_ref[0, 0])
    out_ref[...] = jax.nn.sigmoid(logit).reshape(1, 1, BB)


def _tc_dense(mfu, mfi, mlu, mli, W0, b0, Wp, bp):
    nblk = B // BB
    row_spec = pl.BlockSpec((BB, D), lambda i: (i, 0))
    full = lambda shape: pl.BlockSpec(shape, lambda i: (0,) * len(shape))
    out2d = pl.pallas_call(
        _dense_body,
        grid=(nblk,),
        in_specs=[row_spec, row_spec, row_spec, row_spec,
                  full((2 * D, D)), full((1, D)), full((1, 2 * D)),
                  full((1, 1))],
        out_specs=pl.BlockSpec((1, 1, BB), lambda i: (i, 0, 0)),
        out_shape=jax.ShapeDtypeStruct((nblk, 1, BB), jnp.float32),
    )(mfu, mfi, mlu, mli, W0.T, b0.reshape(1, D), Wp, bp.reshape(1, 1))
    return out2d.reshape(B)


def kernel(user_indices, item_indices, mf_emb_user, mf_emb_item,
           mlp_emb_user, mlp_emb_item, W0, b0, Wp, bp):
    o_mfu, o_mfi, o_mlu, o_mli = _sc_gather(
        user_indices.astype(jnp.int32), item_indices.astype(jnp.int32),
        mf_emb_user.reshape(NBLK, 128), mf_emb_item.reshape(NBLK, 128),
        mlp_emb_user.reshape(NBLK, 128), mlp_emb_item.reshape(NBLK, 128))
    mfu = o_mfu.reshape(B, D)
    mfi = o_mfi.reshape(B, D)
    mlu = o_mlu.reshape(B, D)
    mli = o_mli.reshape(B, D)
    return _tc_dense(mfu, mfi, mlu, mli, W0, b0, Wp, bp)
